# baseline (speedup 1.0000x reference)
import math

import jax
import jax.numpy as jnp
from jax import lax
from jax.experimental import pallas as pl
from jax.experimental.pallas import tpu as pltpu
from jax.experimental.pallas import tpu_sc as plsc

T, D, E, K, FF = 2048, 768, 8, 2, 1536
TM = 128
NP = T * K + E * TM
NB = NP // TM
NBS = T // TM
NBT = NB + NBS
EL = 128
NEG = -3e38
NW = 32


def _router_body(x_ref, rwt_ref, rb_ref, d01_ref, g0_ref, g1_ref, be_ref):
    scale = 1.0 / math.sqrt(D)
    lanes = lax.broadcasted_iota(jnp.int32, (T, EL), 1)
    valid = lanes < E
    lg = jnp.dot(x_ref[:], rwt_ref[:], preferred_element_type=jnp.float32)
    lgv = jnp.where(valid, lg * scale, NEG)
    biased = lgv + rb_ref[:]
    m1 = jnp.max(biased, axis=1, keepdims=True)
    c1 = jnp.logical_and(biased == m1, valid)
    a1 = jnp.min(jnp.where(c1, lanes, EL), axis=1, keepdims=True)
    oh1 = lanes == a1
    b2 = jnp.where(oh1, NEG, biased)
    m2 = jnp.max(b2, axis=1, keepdims=True)
    c2 = jnp.logical_and(b2 == m2, valid)
    a2 = jnp.min(jnp.where(c2, lanes, EL), axis=1, keepdims=True)
    oh2 = lanes == a2
    oh1f = oh1.astype(jnp.float32)
    oh2f = oh2.astype(jnp.float32)
    l1 = jnp.sum(jnp.where(oh1, lgv, 0.0), axis=1, keepdims=True)
    l2 = jnp.sum(jnp.where(oh2, lgv, 0.0), axis=1, keepdims=True)
    mx = jnp.maximum(l1, l2)
    e1 = jnp.exp(l1 - mx)
    e2 = jnp.exp(l2 - mx)
    s = e1 + e2
    g0_ref[:] = jnp.broadcast_to(e1 / s, (T, 16))
    g1_ref[:] = jnp.broadcast_to(e2 / s, (T, 16))
    S = oh1f + oh2f
    tri = (lax.broadcasted_iota(jnp.int32, (T, T), 0)
           > lax.broadcasted_iota(jnp.int32, (T, T), 1)).astype(jnp.bfloat16)
    EXc = lax.dot_general(tri, S.astype(jnp.bfloat16),
                          (((1,), (0,)), ((), ())),
                          preferred_element_type=jnp.float32)
    rank0 = jnp.sum(oh1f * EXc, axis=1, keepdims=True)
    rank1 = jnp.sum(oh2f * (EXc + oh1f), axis=1, keepdims=True)
    cnt = jnp.sum(S, axis=0, keepdims=True)
    nb = (cnt.astype(jnp.int32) + (TM - 1)) // TM
    tri8 = (lax.broadcasted_iota(jnp.int32, (EL, EL), 0)
            < lax.broadcasted_iota(jnp.int32, (EL, EL), 1)).astype(jnp.float32)
    nbf = nb.astype(jnp.float32)
    blk_ex = jnp.dot(nbf, tri8, preferred_element_type=jnp.float32)
    rowoff = TM * blk_ex
    d0 = jnp.sum(oh1f * rowoff, axis=1, keepdims=True) + rank0
    d1 = jnp.sum(oh2f * rowoff, axis=1, keepdims=True) + rank1
    d01_ref[:] = jnp.concatenate([d0, d1], axis=1).astype(jnp.int32)
    inc = blk_ex + nbf
    inc_mat = jnp.broadcast_to(inc, (EL, EL))
    g_rows = lax.broadcasted_iota(jnp.int32, (EL, EL), 0).astype(jnp.float32)
    seg_done = jnp.logical_and(inc_mat <= g_rows, valid[:EL, :])
    be = jnp.minimum(jnp.sum(seg_done.astype(jnp.float32), axis=1,
                             keepdims=True), E - 1)
    be_ref[:] = be.astype(jnp.int32)


def _router(xf, rwt, rb):
    return pl.pallas_call(
        _router_body,
        out_shape=(
            jax.ShapeDtypeStruct((T, 2), jnp.int32),
            jax.ShapeDtypeStruct((T, 16), jnp.float32),
            jax.ShapeDtypeStruct((T, 16), jnp.float32),
            jax.ShapeDtypeStruct((EL, 1), jnp.int32),
        ),
    )(xf, rwt, rb)


_CH = 64


_NR = T * K
_NCH = _NR // (NW * _CH)


def _dispatch_body(x_hbm, tok_hbm, dest_hbm, xs_hbm, tokv, destv, rowsv,
                   sem_i, sem_g, sem_s):
    wid = lax.axis_index("s") * 2 + lax.axis_index("c")
    base = wid * (_NR // NW)
    ci = []
    for c in range(_NCH):
        ci.append(pltpu.async_copy(
            tok_hbm.at[pl.ds(base + c * _CH, _CH)], tokv.at[c], sem_i))
        ci.append(pltpu.async_copy(
            dest_hbm.at[pl.ds(base + c * _CH, _CH)], destv.at[c], sem_i))
    for cp in ci:
        cp.wait()
    cg = [None] * _NCH
    cs = [None] * _NCH
    cg[0] = pltpu.async_copy(x_hbm.at[tokv.at[0]], rowsv.at[0], sem_g)
    for c in range(_NCH):
        if c + 1 < _NCH:
            if c >= 1:
                cs[c - 1].wait()
            cg[c + 1] = pltpu.async_copy(
                x_hbm.at[tokv.at[c + 1]], rowsv.at[(c + 1) % 2], sem_g)
        cg[c].wait()
        cs[c] = pltpu.async_copy(
            rowsv.at[c % 2], xs_hbm.at[destv.at[c]], sem_s)
    for c in range(max(_NCH - 2, 0), _NCH):
        cs[c].wait()


def _dispatch(xf, tok, dest):
    mesh = plsc.VectorSubcoreMesh(core_axis_name="c", subcore_axis_name="s")
    fn = pl.kernel(
        _dispatch_body,
        out_type=jax.ShapeDtypeStruct((NP, D), jnp.float32),
        mesh=mesh,
        scratch_types=[
            pltpu.VMEM((_NCH, _CH), jnp.int32),
            pltpu.VMEM((_NCH, _CH), jnp.int32),
            pltpu.VMEM((2, _CH, D), jnp.float32),
            pltpu.SemaphoreType.DMA,
            pltpu.SemaphoreType.DMA,
            pltpu.SemaphoreType.DMA,
        ],
    )
    return fn(xf, tok, dest)


def _ffn_block(x, w1b, w3b, w2b):
    h1 = lax.dot_general(x, w1b, (((1,), (1,)), ((), ())),
                         preferred_element_type=jnp.float32)
    h3 = lax.dot_general(x, w3b, (((1,), (1,)), ((), ())),
                         preferred_element_type=jnp.float32)
    hidden = h1 * (h3 * jax.nn.sigmoid(h3))
    return lax.dot_general(hidden, w2b, (((1,), (1,)), ((), ())),
                           preferred_element_type=jnp.float32)


def _expert_body(sp_ref, xs_ref, w1_hbm, w3_hbm, w2_hbm, out_ref,
                 w1b, w3b, w2b, s1, s2, s3):
    g = pl.program_id(0)
    fetch = sp_ref[0, g]
    waitf = sp_ref[1, g]
    slot = sp_ref[2, g]
    nxt = sp_ref[3, g]
    cur0 = sp_ref[4, g]

    @pl.when(g == 0)
    def _init():
        pltpu.async_copy(w1_hbm.at[cur0], w1b.at[0], s1).wait()
        pltpu.async_copy(w3_hbm.at[cur0], w3b.at[0], s2).wait()
        pltpu.async_copy(w2_hbm.at[cur0], w2b.at[0], s3).wait()

    @pl.when(fetch == 1)
    def _prefetch():
        pltpu.async_copy(w1_hbm.at[nxt], w1b.at[1 - slot], s1)
        pltpu.async_copy(w3_hbm.at[nxt], w3b.at[1 - slot], s2)
        pltpu.async_copy(w2_hbm.at[nxt], w2b.at[1 - slot], s3)

    @pl.when(waitf == 1)
    def _wait():
        pltpu.make_async_copy(w1_hbm.at[0], w1b.at[slot], s1).wait()
        pltpu.make_async_copy(w3_hbm.at[0], w3b.at[slot], s2).wait()
        pltpu.make_async_copy(w2_hbm.at[0], w2b.at[slot], s3).wait()

    out_ref[:] = _ffn_block(xs_ref[:], w1b[slot], w3b[slot], w2b[slot])


def _experts(sp, xs, w1, w3, w2):
    grid_spec = pltpu.PrefetchScalarGridSpec(
        num_scalar_prefetch=1,
        grid=(NB,),
        in_specs=[
            pl.BlockSpec((TM, D), lambda g, sp: (g, 0)),
            pl.BlockSpec(memory_space=pltpu.HBM),
            pl.BlockSpec(memory_space=pltpu.HBM),
            pl.BlockSpec(memory_space=pltpu.HBM),
        ],
        out_specs=pl.BlockSpec((TM, D), lambda g, sp: (g, 0)),
        scratch_shapes=[
            pltpu.VMEM((2, FF, D), jnp.float32),
            pltpu.VMEM((2, FF, D), jnp.float32),
            pltpu.VMEM((2, D, FF), jnp.float32),
            pltpu.SemaphoreType.DMA,
            pltpu.SemaphoreType.DMA,
            pltpu.SemaphoreType.DMA,
        ],
    )
    return pl.pallas_call(
        _expert_body,
        grid_spec=grid_spec,
        out_shape=jax.ShapeDtypeStruct((NP, D), jnp.float32),
    )(sp, xs, w1, w3, w2)


TMS = 256


def _shared_body(x_ref, sw1_ref, sw3_ref, sw2_ref, out_ref):
    out_ref[:] = _ffn_block(x_ref[:], sw1_ref[:], sw3_ref[:], sw2_ref[:])


def _shared(xf, sw1, sw3, sw2):
    return pl.pallas_call(
        _shared_body,
        grid=(T // TMS,),
        in_specs=[
            pl.BlockSpec((TMS, D), lambda g: (g, 0)),
            pl.BlockSpec((FF, D), lambda g: (0, 0)),
            pl.BlockSpec((FF, D), lambda g: (0, 0)),
            pl.BlockSpec((D, FF), lambda g: (0, 0)),
        ],
        out_specs=pl.BlockSpec((TMS, D), lambda g: (g, 0)),
        out_shape=jax.ShapeDtypeStruct((T, D), jnp.float32),
    )(xf, sw1, sw3, sw2)


_CT = 16


_NCT = T // (NW * _CT)


def _combine_body(eo_hbm, sh_hbm, d01_hbm, g0_hbm, g1_hbm, out_hbm,
                  dv, g0v, g1v, shv, abv, sem_i, sem_ab, sem_o):
    wid = lax.axis_index("s") * 2 + lax.axis_index("c")
    base0 = wid * (T // NW)
    ci = []
    for c in range(_NCT):
        base = base0 + c * _CT
        ci.append(pltpu.async_copy(
            d01_hbm.at[pl.ds(base * K, _CT * K)], dv.at[c], sem_i))
        ci.append(pltpu.async_copy(
            g0_hbm.at[pl.ds(base, _CT)], g0v.at[c], sem_i))
        ci.append(pltpu.async_copy(
            g1_hbm.at[pl.ds(base, _CT)], g1v.at[c], sem_i))
        ci.append(pltpu.async_copy(
            sh_hbm.at[pl.ds(base, _CT)], shv.at[c], sem_i))
    for cp in ci:
        cp.wait()
    cg = [None] * _NCT
    co = [None] * _NCT
    cg[0] = pltpu.async_copy(eo_hbm.at[dv.at[0]], abv.at[0], sem_ab)
    for c in range(_NCT):
        slot = c % 2
        if c + 1 < _NCT:
            cg[c + 1] = pltpu.async_copy(
                eo_hbm.at[dv.at[c + 1]], abv.at[(c + 1) % 2], sem_ab)
        cg[c].wait()

        @plsc.parallel_loop(0, _CT, 1)
        def tok_body(t):
            g0b = g0v[c, t, :]
            g1b = g1v[c, t, :]

            @plsc.parallel_loop(0, D // 16, 1, unroll=8)
            def col_body(cc):
                off = cc * 16
                shv[c, t, pl.ds(off, 16)] = (
                    g0b * abv[slot, 2 * t, pl.ds(off, 16)]
                    + g1b * abv[slot, 2 * t + 1, pl.ds(off, 16)]
                    + shv[c, t, pl.ds(off, 16)])
        co[c] = pltpu.async_copy(
            shv.at[c], out_hbm.at[pl.ds(base0 + c * _CT, _CT)], sem_o)
    for c in range(_NCT):
        co[c].wait()


def _combine(eo, sh, d01, g0, g1):
    mesh = plsc.VectorSubcoreMesh(core_axis_name="c", subcore_axis_name="s")
    fn = pl.kernel(
        _combine_body,
        out_type=jax.ShapeDtypeStruct((T, D), jnp.float32),
        mesh=mesh,
        scratch_types=[
            pltpu.VMEM((_NCT, _CT * K), jnp.int32),
            pltpu.VMEM((_NCT, _CT, 16), jnp.float32),
            pltpu.VMEM((_NCT, _CT, 16), jnp.float32),
            pltpu.VMEM((_NCT, _CT, D), jnp.float32),
            pltpu.VMEM((2, _CT * K, D), jnp.float32),
            pltpu.SemaphoreType.DMA,
            pltpu.SemaphoreType.DMA,
            pltpu.SemaphoreType.DMA,
        ],
    )
    return fn(eo, sh, d01, g0, g1)


def kernel(x, router_w, router_b, w1, w3, w2, sw1, sw3, sw2):
    Bsz, Sl, Dm = x.shape
    xf = x.reshape(-1, Dm)
    rwt = jnp.zeros((D, EL), jnp.float32).at[:, :E].set(router_w.T)
    rb = jnp.zeros((1, EL), jnp.float32).at[0, :E].set(router_b)

    d01, g0, g1, be_col = _router(xf, rwt, rb)
    be = be_col[:NB, 0]
    dest = d01.reshape(T * K)
    tok = jnp.repeat(jnp.arange(T, dtype=jnp.int32), K)

    first = jnp.concatenate(
        [jnp.ones((1,), jnp.bool_), be[1:] != be[:-1]])
    seg_id = jnp.cumsum(first.astype(jnp.int32)) - 1
    nseg = seg_id[-1] + 1
    expert_of_seg = jnp.zeros((NB,), jnp.int32).at[seg_id].set(be)
    nxt_expert = expert_of_seg[jnp.minimum(seg_id + 1, NB - 1)]
    sp = jnp.stack([
        (first & (seg_id + 1 < nseg)).astype(jnp.int32),
        (first & (jnp.arange(NB) > 0)).astype(jnp.int32),
        seg_id % 2,
        nxt_expert,
        jnp.full((NB,), be[0], jnp.int32),
    ])

    xs = _dispatch(xf, tok, dest)
    eo = _experts(sp, xs, w1, w3, w2)
    sh = _shared(xf, sw1, sw3, sw2)
    out = _combine(eo, sh, dest, g0, g1)
    return out.reshape(Bsz, Sl, Dm)


# speedup vs baseline: 1.0061x; 1.0061x over previous
import math

import jax
import jax.numpy as jnp
from jax import lax
from jax.experimental import pallas as pl
from jax.experimental.pallas import tpu as pltpu
from jax.experimental.pallas import tpu_sc as plsc

T, D, E, K, FF = 2048, 768, 8, 2, 1536
TM = 128
NP = T * K + E * TM
NB = NP // TM
NBS = T // TM
NBT = NB + NBS
EL = 128
NEG = -3e38
NW = 32


def _router_body(x_ref, rwt_ref, rb_ref, d01_ref, g0_ref, g1_ref, be_ref):
    scale = 1.0 / math.sqrt(D)
    lanes = lax.broadcasted_iota(jnp.int32, (T, EL), 1)
    valid = lanes < E
    lg = jnp.dot(x_ref[:], rwt_ref[:], preferred_element_type=jnp.float32)
    lgv = jnp.where(valid, lg * scale, NEG)
    biased = lgv + rb_ref[:]
    m1 = jnp.max(biased, axis=1, keepdims=True)
    c1 = jnp.logical_and(biased == m1, valid)
    a1 = jnp.min(jnp.where(c1, lanes, EL), axis=1, keepdims=True)
    oh1 = lanes == a1
    b2 = jnp.where(oh1, NEG, biased)
    m2 = jnp.max(b2, axis=1, keepdims=True)
    c2 = jnp.logical_and(b2 == m2, valid)
    a2 = jnp.min(jnp.where(c2, lanes, EL), axis=1, keepdims=True)
    oh2 = lanes == a2
    oh1f = oh1.astype(jnp.float32)
    oh2f = oh2.astype(jnp.float32)
    l1 = jnp.sum(jnp.where(oh1, lgv, 0.0), axis=1, keepdims=True)
    l2 = jnp.sum(jnp.where(oh2, lgv, 0.0), axis=1, keepdims=True)
    mx = jnp.maximum(l1, l2)
    e1 = jnp.exp(l1 - mx)
    e2 = jnp.exp(l2 - mx)
    s = e1 + e2
    g0_ref[:] = jnp.broadcast_to(e1 / s, (T, 16))
    g1_ref[:] = jnp.broadcast_to(e2 / s, (T, 16))
    S = oh1f + oh2f
    tri = (lax.broadcasted_iota(jnp.int32, (T, T), 0)
           > lax.broadcasted_iota(jnp.int32, (T, T), 1)).astype(jnp.bfloat16)
    EXc = lax.dot_general(tri, S.astype(jnp.bfloat16),
                          (((1,), (0,)), ((), ())),
                          preferred_element_type=jnp.float32)
    rank0 = jnp.sum(oh1f * EXc, axis=1, keepdims=True)
    rank1 = jnp.sum(oh2f * (EXc + oh1f), axis=1, keepdims=True)
    cnt = jnp.sum(S, axis=0, keepdims=True)
    nb = (cnt.astype(jnp.int32) + (TM - 1)) // TM
    tri8 = (lax.broadcasted_iota(jnp.int32, (EL, EL), 0)
            < lax.broadcasted_iota(jnp.int32, (EL, EL), 1)).astype(jnp.float32)
    nbf = nb.astype(jnp.float32)
    blk_ex = jnp.dot(nbf, tri8, preferred_element_type=jnp.float32)
    rowoff = TM * blk_ex
    d0 = jnp.sum(oh1f * rowoff, axis=1, keepdims=True) + rank0
    d1 = jnp.sum(oh2f * rowoff, axis=1, keepdims=True) + rank1
    d01_ref[:] = jnp.concatenate([d0, d1], axis=1).astype(jnp.int32)
    inc = blk_ex + nbf
    inc_mat = jnp.broadcast_to(inc, (EL, EL))
    g_rows = lax.broadcasted_iota(jnp.int32, (EL, EL), 0).astype(jnp.float32)
    seg_done = jnp.logical_and(inc_mat <= g_rows, valid[:EL, :])
    be = jnp.minimum(jnp.sum(seg_done.astype(jnp.float32), axis=1,
                             keepdims=True), E - 1)
    be_ref[:] = be.astype(jnp.int32)


def _router(xf, rwt, rb):
    return pl.pallas_call(
        _router_body,
        out_shape=(
            jax.ShapeDtypeStruct((T, 2), jnp.int32),
            jax.ShapeDtypeStruct((T, 16), jnp.float32),
            jax.ShapeDtypeStruct((T, 16), jnp.float32),
            jax.ShapeDtypeStruct((EL, 1), jnp.int32),
        ),
    )(xf, rwt, rb)


_CH = 64


_NR = T * K
_NCH = _NR // (NW * _CH)


def _dispatch_body(x_hbm, tok_hbm, dest_hbm, xs_hbm, tokv, destv, rowsv,
                   sem_i, sem_g, sem_s):
    wid = lax.axis_index("s") * 2 + lax.axis_index("c")
    base = wid * (_NR // NW)
    ci = []
    for c in range(_NCH):
        ci.append(pltpu.async_copy(
            tok_hbm.at[pl.ds(base + c * _CH, _CH)], tokv.at[c], sem_i))
        ci.append(pltpu.async_copy(
            dest_hbm.at[pl.ds(base + c * _CH, _CH)], destv.at[c], sem_i))
    for cp in ci:
        cp.wait()
    cg = [None] * _NCH
    cs = [None] * _NCH
    cg[0] = pltpu.async_copy(x_hbm.at[tokv.at[0]], rowsv.at[0], sem_g)
    for c in range(_NCH):
        if c + 1 < _NCH:
            if c >= 1:
                cs[c - 1].wait()
            cg[c + 1] = pltpu.async_copy(
                x_hbm.at[tokv.at[c + 1]], rowsv.at[(c + 1) % 2], sem_g)
        cg[c].wait()
        cs[c] = pltpu.async_copy(
            rowsv.at[c % 2], xs_hbm.at[destv.at[c]], sem_s)
    for c in range(max(_NCH - 2, 0), _NCH):
        cs[c].wait()


def _dispatch(xf, tok, dest):
    mesh = plsc.VectorSubcoreMesh(core_axis_name="c", subcore_axis_name="s")
    fn = pl.kernel(
        _dispatch_body,
        out_type=jax.ShapeDtypeStruct((NP, D), jnp.float32),
        mesh=mesh,
        scratch_types=[
            pltpu.VMEM((_NCH, _CH), jnp.int32),
            pltpu.VMEM((_NCH, _CH), jnp.int32),
            pltpu.VMEM((2, _CH, D), jnp.float32),
            pltpu.SemaphoreType.DMA,
            pltpu.SemaphoreType.DMA,
            pltpu.SemaphoreType.DMA,
        ],
    )
    return fn(xf, tok, dest)


def _ffn_block(x, w1b, w3b, w2b):
    h1 = lax.dot_general(x, w1b, (((1,), (1,)), ((), ())),
                         preferred_element_type=jnp.float32)
    h3 = lax.dot_general(x, w3b, (((1,), (1,)), ((), ())),
                         preferred_element_type=jnp.float32)
    hidden = h1 * (h3 * jax.nn.sigmoid(h3))
    return lax.dot_general(hidden, w2b, (((1,), (1,)), ((), ())),
                           preferred_element_type=jnp.float32)


def _expert_body(sp_ref, xs_ref, w1_hbm, w3_hbm, w2_hbm, out_ref,
                 w1b, w3b, w2b, s1, s2, s3):
    g = pl.program_id(0)
    fetch = sp_ref[0, g]
    waitf = sp_ref[1, g]
    slot = sp_ref[2, g]
    nxt = sp_ref[3, g]
    cur0 = sp_ref[4, g]

    @pl.when(g == 0)
    def _init():
        pltpu.async_copy(w1_hbm.at[cur0], w1b.at[0], s1).wait()
        pltpu.async_copy(w3_hbm.at[cur0], w3b.at[0], s2).wait()
        pltpu.async_copy(w2_hbm.at[cur0], w2b.at[0], s3).wait()

    @pl.when(fetch == 1)
    def _prefetch():
        pltpu.async_copy(w1_hbm.at[nxt], w1b.at[1 - slot], s1)
        pltpu.async_copy(w3_hbm.at[nxt], w3b.at[1 - slot], s2)
        pltpu.async_copy(w2_hbm.at[nxt], w2b.at[1 - slot], s3)

    @pl.when(waitf == 1)
    def _wait():
        pltpu.make_async_copy(w1_hbm.at[0], w1b.at[slot], s1).wait()
        pltpu.make_async_copy(w3_hbm.at[0], w3b.at[slot], s2).wait()
        pltpu.make_async_copy(w2_hbm.at[0], w2b.at[slot], s3).wait()

    out_ref[:] = _ffn_block(xs_ref[:], w1b[slot], w3b[slot], w2b[slot])


def _experts(sp, xs, w1, w3, w2):
    grid_spec = pltpu.PrefetchScalarGridSpec(
        num_scalar_prefetch=1,
        grid=(NB,),
        in_specs=[
            pl.BlockSpec((TM, D), lambda g, sp: (g, 0)),
            pl.BlockSpec(memory_space=pltpu.HBM),
            pl.BlockSpec(memory_space=pltpu.HBM),
            pl.BlockSpec(memory_space=pltpu.HBM),
        ],
        out_specs=pl.BlockSpec((TM, D), lambda g, sp: (g, 0)),
        scratch_shapes=[
            pltpu.VMEM((2, FF, D), jnp.float32),
            pltpu.VMEM((2, FF, D), jnp.float32),
            pltpu.VMEM((2, D, FF), jnp.float32),
            pltpu.SemaphoreType.DMA,
            pltpu.SemaphoreType.DMA,
            pltpu.SemaphoreType.DMA,
        ],
    )
    return pl.pallas_call(
        _expert_body,
        grid_spec=grid_spec,
        out_shape=jax.ShapeDtypeStruct((NP, D), jnp.float32),
    )(sp, xs, w1, w3, w2)


TMS = 256


def _shared_body(x_ref, sw1_ref, sw3_ref, sw2_ref, out_ref):
    out_ref[:] = _ffn_block(x_ref[:], sw1_ref[:], sw3_ref[:], sw2_ref[:])


def _shared(xf, sw1, sw3, sw2):
    return pl.pallas_call(
        _shared_body,
        grid=(T // TMS,),
        in_specs=[
            pl.BlockSpec((TMS, D), lambda g: (g, 0)),
            pl.BlockSpec((FF, D), lambda g: (0, 0)),
            pl.BlockSpec((FF, D), lambda g: (0, 0)),
            pl.BlockSpec((D, FF), lambda g: (0, 0)),
        ],
        out_specs=pl.BlockSpec((TMS, D), lambda g: (g, 0)),
        out_shape=jax.ShapeDtypeStruct((T, D), jnp.float32),
    )(xf, sw1, sw3, sw2)


_CT = 16


_NCT = T // (NW * _CT)


def _combine_body(eo_hbm, sh_hbm, d01_hbm, g0_hbm, g1_hbm, out_hbm,
                  dv, g0v, g1v, shv, abv, sem_i, sem_ab, sem_o):
    wid = lax.axis_index("s") * 2 + lax.axis_index("c")
    base0 = wid * (T // NW)
    ci = []
    for c in range(_NCT):
        base = base0 + c * _CT
        ci.append(pltpu.async_copy(
            d01_hbm.at[pl.ds(base * K, _CT * K)], dv.at[c], sem_i))
        ci.append(pltpu.async_copy(
            g0_hbm.at[pl.ds(base, _CT)], g0v.at[c], sem_i))
        ci.append(pltpu.async_copy(
            g1_hbm.at[pl.ds(base, _CT)], g1v.at[c], sem_i))
        ci.append(pltpu.async_copy(
            sh_hbm.at[pl.ds(base, _CT)], shv.at[c], sem_i))
    for cp in ci:
        cp.wait()
    cg = [None] * _NCT
    co = [None] * _NCT
    cg[0] = pltpu.async_copy(eo_hbm.at[dv.at[0]], abv.at[0], sem_ab)
    for c in range(_NCT):
        slot = c % 2
        if c + 1 < _NCT:
            cg[c + 1] = pltpu.async_copy(
                eo_hbm.at[dv.at[c + 1]], abv.at[(c + 1) % 2], sem_ab)
        cg[c].wait()

        @plsc.parallel_loop(0, _CT, 1)
        def tok_body(t):
            g0b = g0v[c, t, :]
            g1b = g1v[c, t, :]

            @plsc.parallel_loop(0, D // 16, 1, unroll=4)
            def col_body(cc):
                off = cc * 16
                shv[c, t, pl.ds(off, 16)] = (
                    g0b * abv[slot, 2 * t, pl.ds(off, 16)]
                    + g1b * abv[slot, 2 * t + 1, pl.ds(off, 16)]
                    + shv[c, t, pl.ds(off, 16)])
        co[c] = pltpu.async_copy(
            shv.at[c], out_hbm.at[pl.ds(base0 + c * _CT, _CT)], sem_o)
    for c in range(_NCT):
        co[c].wait()


def _combine(eo, sh, d01, g0, g1):
    mesh = plsc.VectorSubcoreMesh(core_axis_name="c", subcore_axis_name="s")
    fn = pl.kernel(
        _combine_body,
        out_type=jax.ShapeDtypeStruct((T, D), jnp.float32),
        mesh=mesh,
        scratch_types=[
            pltpu.VMEM((_NCT, _CT * K), jnp.int32),
            pltpu.VMEM((_NCT, _CT, 16), jnp.float32),
            pltpu.VMEM((_NCT, _CT, 16), jnp.float32),
            pltpu.VMEM((_NCT, _CT, D), jnp.float32),
            pltpu.VMEM((2, _CT * K, D), jnp.float32),
            pltpu.SemaphoreType.DMA,
            pltpu.SemaphoreType.DMA,
            pltpu.SemaphoreType.DMA,
        ],
    )
    return fn(eo, sh, d01, g0, g1)


def kernel(x, router_w, router_b, w1, w3, w2, sw1, sw3, sw2):
    Bsz, Sl, Dm = x.shape
    xf = x.reshape(-1, Dm)
    rwt = jnp.zeros((D, EL), jnp.float32).at[:, :E].set(router_w.T)
    rb = jnp.zeros((1, EL), jnp.float32).at[0, :E].set(router_b)

    d01, g0, g1, be_col = _router(xf, rwt, rb)
    be = be_col[:NB, 0]
    dest = d01.reshape(T * K)
    tok = jnp.repeat(jnp.arange(T, dtype=jnp.int32), K)

    first = jnp.concatenate(
        [jnp.ones((1,), jnp.bool_), be[1:] != be[:-1]])
    seg_id = jnp.cumsum(first.astype(jnp.int32)) - 1
    nseg = seg_id[-1] + 1
    expert_of_seg = jnp.zeros((NB,), jnp.int32).at[seg_id].set(be)
    nxt_expert = expert_of_seg[jnp.minimum(seg_id + 1, NB - 1)]
    sp = jnp.stack([
        (first & (seg_id + 1 < nseg)).astype(jnp.int32),
        (first & (jnp.arange(NB) > 0)).astype(jnp.int32),
        seg_id % 2,
        nxt_expert,
        jnp.full((NB,), be[0], jnp.int32),
    ])

    xs = _dispatch(xf, tok, dest)
    eo = _experts(sp, xs, w1, w3, w2)
    sh = _shared(xf, sw1, sw3, sw2)
    out = _combine(eo, sh, dest, g0, g1)
    return out.reshape(Bsz, Sl, Dm)
